# Initial kernel scaffold; baseline (speedup 1.0000x reference)
import jax
import jax.numpy as jnp
from jax.experimental import pallas as pl


def kernel(fushed_features, input_data, adj_idx, centroids, W1, b1, W2, b2, W3, b3, Wg, bg, gamma, beta, cluster_results):
    raise NotImplementedError("write your pallas kernel here")


# speedup vs baseline: 5.0356x; 5.0356x over previous
import jax
import jax.numpy as jnp
from jax.experimental import pallas as pl
from functools import partial

B, N, C, K, D, T, SOUT = 64, 325, 6, 20, 128, 12, 12
MARGIN = 1.0

_dot = partial(jax.lax.dot_general, dimension_numbers=(((1,), (0,)), ((), ())),
               preferred_element_type=jnp.float32)
_dott = partial(jax.lax.dot_general, dimension_numbers=(((1,), (1,)), ((), ())),
                preferred_element_type=jnp.float32)
_dott_hi = partial(jax.lax.dot_general, dimension_numbers=(((1,), (1,)), ((), ())),
                   preferred_element_type=jnp.float32,
                   precision=jax.lax.Precision.HIGHEST)


def _stats_kernel(ff_ref, mean_ref, var_ref):
    x = ff_ref[...]
    m = x.mean(axis=2, keepdims=True).mean(axis=0, keepdims=True)
    xc = x - m
    v = (xc * xc).mean(axis=2, keepdims=True).mean(axis=0, keepdims=True)
    mean_ref[...] = m
    var_ref[...] = v


def _cdist(x1, x2, nd):
    n1 = x1.shape[0]
    n2 = x2.shape[0]
    adj = x1.mean(axis=0, keepdims=True)
    a = x1 - adj
    b = x2 - adj
    an = (a * a).sum(axis=1, keepdims=True)
    bn = (b * b).sum(axis=1, keepdims=True)
    x1_ = jnp.concatenate([-2.0 * a, an, jnp.ones((n1, 1), jnp.float32)], axis=1)
    x2_ = jnp.concatenate([b, jnp.ones((n2, 1), jnp.float32), bn], axis=1)
    res = _dott(x1_, x2_)
    return jnp.sqrt(jnp.clip(res, 1e-30))


def _main_kernel(ff_ref, x_ref, idx_ref, cen_ref, w1_ref, b1_ref, w2_ref,
                 b2_ref, w3_ref, b3_ref, wg_ref, bg_ref, gam_ref, bet_ref,
                 cr_ref, mean_ref, var_ref,
                 out_ref, l1_ref, l2_ref, l3_ref):
    b = pl.program_id(0)
    ff = ff_ref[0]
    x = x_ref[0]
    idx = idx_ref[0]
    cen = cen_ref[0]
    mean = mean_ref[0]
    var = var_ref[0]

    ffn = (ff - mean) / jnp.sqrt(var + 1e-5)
    ffn = ffn * gam_ref[...] + bet_ref[...]

    h = jax.nn.relu(_dot(cen, w1_ref[...]) + b1_ref[...])
    h = jax.nn.relu(_dot(h, w2_ref[...]) + b2_ref[...])
    cf = h + jax.nn.relu(_dot(cen, w3_ref[...]) + b3_ref[...])

    simi = _cdist(ffn, cen, D)
    sm = jax.nn.softmax(simi, axis=-1)
    sm = jnp.where(sm < 1.0 / C, 0.0, sm)

    cols = jax.lax.broadcasted_iota(jnp.int32, (N * K, N), 1)
    onehot = (idx == cols).astype(jnp.float32)
    simi_top = _dot(onehot, sm)
    xt = _dot(onehot, x)
    st3 = simi_top.reshape(N, K, C)

    outs = []
    for i in range(C):
        w = jax.nn.relu(_dot(xt, wg_ref[i]) + bg_ref[i:i + 1, :])
        si = simi_top[:, i:i + 1]
        wsum = (w * si).reshape(N, K, SOUT).sum(axis=1)
        cnt = st3[:, :, i:i + 1].sum(axis=1)
        cnt = jnp.where(cnt == 0.0, 1.0, cnt)
        outs.append(wsum / cnt)
    out_ref[0] = jnp.concatenate(outs, axis=-1)

    cr = jax.nn.softmax(cr_ref[...], axis=-1)
    logp = jax.nn.log_softmax(st3, axis=-1)
    t = cr * (jnp.log(cr) - logp)
    kl_b = t.sum(axis=2).sum(axis=1, keepdims=True).sum(
        axis=0, keepdims=True) / (B * N)

    cs = jax.nn.softmax(cf, axis=-1)
    lcs = jnp.log(cs)
    eT = _dott(jnp.ones((1, D), jnp.float32), cs * lcs)
    dkl = eT - _dott(lcs, cs)
    offd = 1.0 - jnp.eye(C, dtype=jnp.float32)
    ckl_b = (offd * dkl).sum(axis=1, keepdims=True).sum(
        axis=0, keepdims=True) / B

    dist = _cdist(cen, cen, D)
    diff = jnp.clip(offd * MARGIN - dist, 0.0, None) ** 2
    ccl_b = diff.sum(axis=1, keepdims=True).sum(axis=0, keepdims=True) / B

    @pl.when(b == 0)
    def _():
        l1_ref[...] = kl_b
        l2_ref[...] = ccl_b
        l3_ref[...] = ckl_b

    @pl.when(b != 0)
    def _():
        l1_ref[...] += kl_b
        l2_ref[...] += ccl_b
        l3_ref[...] += ckl_b


def kernel(fushed_features, input_data, adj_idx, centroids, W1, b1, W2, b2,
           W3, b3, Wg, bg, gamma, beta, cluster_results):
    mean, var = pl.pallas_call(
        _stats_kernel,
        out_shape=(jax.ShapeDtypeStruct((1, N, 1), jnp.float32),
                   jax.ShapeDtypeStruct((1, N, 1), jnp.float32)),
    )(fushed_features)

    x = input_data[:, 0]
    idx_flat = adj_idx.reshape(B, N * K, 1)
    rep = lambda *blk: pl.BlockSpec(blk, lambda b: (0,) * len(blk))
    perb = lambda *blk: pl.BlockSpec(blk, lambda b: (b,) + (0,) * (len(blk) - 1))

    out, l1, l2, l3 = pl.pallas_call(
        _main_kernel,
        grid=(B,),
        in_specs=[
            perb(1, N, D),
            perb(1, N, T),
            perb(1, N * K, 1),
            perb(1, C, D),
            rep(D, D), rep(1, D),
            rep(D, D), rep(1, D),
            rep(D, D), rep(1, D),
            rep(C, T, SOUT), rep(C, SOUT),
            rep(N, 1), rep(N, 1),
            rep(N, K, C),
            rep(1, N, 1), rep(1, N, 1),
        ],
        out_specs=[
            pl.BlockSpec((1, N, C * SOUT), lambda b: (b, 0, 0)),
            pl.BlockSpec((1, 1), lambda b: (0, 0)),
            pl.BlockSpec((1, 1), lambda b: (0, 0)),
            pl.BlockSpec((1, 1), lambda b: (0, 0)),
        ],
        out_shape=(
            jax.ShapeDtypeStruct((B, N, C * SOUT), jnp.float32),
            jax.ShapeDtypeStruct((1, 1), jnp.float32),
            jax.ShapeDtypeStruct((1, 1), jnp.float32),
            jax.ShapeDtypeStruct((1, 1), jnp.float32),
        ),
    )(fushed_features, x, idx_flat, centroids, W1, b1[None, :], W2, b2[None, :],
      W3, b3[None, :], Wg, bg, gamma[:, None], beta[:, None], cluster_results,
      mean, var)

    updated_input = out.reshape(B, N, C, SOUT)
    return (updated_input, l1[0, 0], l2[0, 0], -1.0 * l3[0, 0])
